# Initial kernel scaffold; baseline (speedup 1.0000x reference)
import functools

import jax
import jax.numpy as jnp
from jax import lax
from jax.experimental import pallas as pl
from jax.experimental.pallas import tpu as pltpu
from jax.experimental.pallas import tpu_sc as plsc

N = 10000
NR = 10016
NG = 512
E = 320000
NCHUNK = 79
EP = 32 * NCHUNK * 128
OUT = 128
BN_EPS = 1e-5
NEG = -jnp.inf

_HIGH = jax.lax.Precision.HIGHEST


def _dot(a, b):
    return jnp.dot(a, b, precision=_HIGH, preferred_element_type=jnp.float32)


def _make_aggregate(W):
    ZR = 313
    mesh = plsc.VectorSubcoreMesh(core_axis_name="c", subcore_axis_name="s")

    @functools.partial(
        pl.kernel,
        out_type=jax.ShapeDtypeStruct((2, NR, W), jnp.float32),
        mesh=mesh,
        scratch_types=[
            pltpu.VMEM_SHARED((NR, W), jnp.float32),
            pltpu.VMEM((ZR, W), jnp.float32),
            pltpu.VMEM((NCHUNK, 128), jnp.int32),
            pltpu.VMEM((NCHUNK, 128), jnp.int32),
            pltpu.VMEM((128, W), jnp.float32),
            pltpu.SemaphoreType.DMA,
        ],
    )
    def agg(y_hbm, src_hbm, dst_hbm, out_hbm, accum, zbuf, src_v, dst_v,
            rows_v, sem):
        c = lax.axis_index("c")
        s = lax.axis_index("s")
        w = c * 16 + s
        zeros16 = jnp.zeros((16,), jnp.float32)

        def zb(i, carry):
            for k in range(W // 16):
                zbuf[i, pl.ds(k * 16, 16)] = zeros16
            return carry

        lax.fori_loop(0, ZR, zb, 0)
        pltpu.sync_copy(src_hbm.at[w], src_v)
        pltpu.sync_copy(dst_hbm.at[w], dst_v)
        pltpu.sync_copy(zbuf, accum.at[pl.ds(s * 626, ZR)])
        pltpu.sync_copy(zbuf, accum.at[pl.ds(s * 626 + ZR, ZR)])
        plsc.subcore_barrier()

        def body(j, carry):
            pltpu.async_copy(y_hbm.at[src_v.at[j]], rows_v, sem).wait()
            pltpu.sync_copy(rows_v, accum.at[dst_v.at[j]], add=True)
            return carry

        lax.fori_loop(0, NCHUNK, body, 0)
        plsc.subcore_barrier()
        pltpu.sync_copy(accum.at[pl.ds(s * 626, 626)],
                        out_hbm.at[c, pl.ds(s * 626, 626)])

    return agg


_agg80 = _make_aggregate(80)
_agg128 = _make_aggregate(128)


def _bn_cols(u, g, b):
    m = jnp.mean(u, axis=0)
    v = jnp.mean((u - m) * (u - m), axis=0)
    return (u - m) / jnp.sqrt(v + BN_EPS) * g + b


def _gin_layer_body(x_ref, p_ref, w1_ref, b1_ref, w2_ref, b2_ref, g_ref,
                    b_ref, out_ref):
    h = x_ref[0, :N, :] + p_ref[0, :N, :] + p_ref[1, :N, :]
    a = jnp.maximum(_dot(h, w1_ref[...]) + b1_ref[...], 0.0)
    z = _dot(a, w2_ref[...]) + b2_ref[...]
    u = jnp.maximum(z, 0.0)
    un = _bn_cols(u, g_ref[...], b_ref[...])
    out_ref[...] = jnp.concatenate(
        [un, jnp.zeros((NR - N, OUT), jnp.float32)], axis=0)


def _gin_layer(x_pad, p, w1, b1, w2, b2, bn_g, bn_b):
    return pl.pallas_call(
        _gin_layer_body,
        out_shape=jax.ShapeDtypeStruct((NR, OUT), jnp.float32),
    )(x_pad[None], p, w1, b1, w2, b2, bn_g, bn_b)


def _final_body(x_ref, p_ref, w1_ref, b1_ref, w2_ref, b2_ref, g_ref, b_ref,
                batch_ref, fcw_ref, fcb_ref, out_ref):
    h = x_ref[0, :N, :] + p_ref[0, :N, :] + p_ref[1, :N, :]
    a = jnp.maximum(_dot(h, w1_ref[...]) + b1_ref[...], 0.0)
    z = _dot(a, w2_ref[...]) + b2_ref[...]
    u = jnp.maximum(z, 0.0)
    un = _bn_cols(u, g_ref[...], b_ref[...])

    bid = batch_ref[...]
    un = jnp.concatenate(
        [un, jnp.full((NR - N, OUT), NEG, jnp.float32)], axis=0)
    s = 1
    while s < NR:
        u_sh = jnp.concatenate(
            [jnp.full((s, OUT), NEG, jnp.float32), un[:-s, :]], axis=0)
        b_sh = jnp.concatenate(
            [jnp.full((s, 1), -1, jnp.int32), bid[:-s, :]], axis=0)
        un = jnp.where(b_sh == bid, jnp.maximum(un, u_sh), un)
        s *= 2
    is_end = jnp.concatenate(
        [bid[:-1, :] != bid[1:, :], jnp.full((1, 1), True)], axis=0)
    gids = lax.broadcasted_iota(jnp.int32, (NR, NG), 1)
    onehot = (bid == gids).astype(jnp.float32)
    sel = onehot * is_end.astype(jnp.float32)
    pooled = lax.dot_general(sel, un, (((0,), (0,)), ((), ())),
                             precision=_HIGH,
                             preferred_element_type=jnp.float32)
    counts = jnp.sum(onehot, axis=0)
    pooled = jnp.where(counts[:, None] > 0, pooled, NEG)
    out_ref[...] = jnp.maximum(_dot(pooled, fcw_ref[...]) + fcb_ref[...], 0.0)


def _final_stage(x_pad, p, w1, b1, w2, b2, bn_g, bn_b, batch_pad, fc_w, fc_b):
    return pl.pallas_call(
        _final_body,
        out_shape=jax.ShapeDtypeStruct((NG, OUT), jnp.float32),
    )(x_pad[None], p, w1, b1, w2, b2, bn_g, bn_b, batch_pad, fc_w, fc_b)


def _branch_body(mic_ref, dis_ref, mw_ref, mb_ref, dw_ref, db_ref, mg_ref,
                 mbb_ref, dg_ref, dbb_ref, mic_out, dis_out):
    xm = jnp.maximum(_dot(mic_ref[...], mw_ref[...]) + mb_ref[...], 0.0)
    mic_out[...] = _bn_cols(xm, mg_ref[...], mbb_ref[...])
    xd = jnp.maximum(_dot(dis_ref[...], dw_ref[...]) + db_ref[...], 0.0)
    dis_out[...] = _bn_cols(xd, dg_ref[...], dbb_ref[...])


def _branches(mic_feature, dis_feature, mic_w, mic_b, dis_w, dis_b,
              bn_mic_g, bn_mic_b, bn_dis_g, bn_dis_b):
    return pl.pallas_call(
        _branch_body,
        out_shape=(jax.ShapeDtypeStruct((4096, OUT), jnp.float32),
                   jax.ShapeDtypeStruct((4096, OUT), jnp.float32)),
    )(mic_feature, dis_feature, mic_w, mic_b, dis_w, dis_b,
      bn_mic_g, bn_mic_b, bn_dis_g, bn_dis_b)


def kernel(x, edge_index, batch, mic_feature, dis_feature,
           conv1_w1, conv1_b1, conv1_w2, conv1_b2,
           conv2_w1, conv2_b1, conv2_w2, conv2_b2,
           conv3_w1, conv3_b1, conv3_w2, conv3_b2,
           bn1_g, bn1_b, bn2_g, bn2_b, bn3_g, bn3_b,
           bn_dis_g, bn_dis_b, bn_mic_g, bn_mic_b,
           fc_w, fc_b, dis_w, dis_b, mic_w, mic_b):
    pad = EP - E
    src3 = jnp.concatenate(
        [edge_index[0], jnp.zeros((pad,), jnp.int32)]).reshape(32, NCHUNK, 128)
    dst3 = jnp.concatenate(
        [edge_index[1], jnp.full((pad,), N, jnp.int32)]).reshape(32, NCHUNK, 128)
    x_pad = jnp.pad(x, ((0, NR - N), (0, 2)))
    w1_pad = jnp.pad(conv1_w1, ((0, 2), (0, 0)))
    batch_pad = jnp.concatenate(
        [batch, jnp.full((NR - N,), NG, jnp.int32)]).reshape(NR, 1)

    p1 = _agg80(x_pad, src3, dst3)
    u1 = _gin_layer(x_pad, p1, w1_pad, conv1_b1, conv1_w2, conv1_b2,
                    bn1_g, bn1_b)
    p2 = _agg128(u1, src3, dst3)
    u2 = _gin_layer(u1, p2, conv2_w1, conv2_b1, conv2_w2, conv2_b2,
                    bn2_g, bn2_b)
    p3 = _agg128(u2, src3, dst3)
    x_d = _final_stage(u2, p3, conv3_w1, conv3_b1, conv3_w2, conv3_b2,
                       bn3_g, bn3_b, batch_pad, fc_w, fc_b)
    x_mic, x_dis = _branches(mic_feature, dis_feature, mic_w, mic_b,
                             dis_w, dis_b, bn_mic_g, bn_mic_b,
                             bn_dis_g, bn_dis_b)
    return (x_d, x_mic, x_dis)


# speedup vs baseline: 1.8900x; 1.8900x over previous
import functools

import jax
import jax.numpy as jnp
from jax import lax
from jax.experimental import pallas as pl
from jax.experimental.pallas import tpu as pltpu
from jax.experimental.pallas import tpu_sc as plsc

N = 10000
NR = 10112
H = NR // 2
AR = 5120
PT = AR // 16
NG = 512
E = 320000
NCHUNK = 160
SEC = 8
EP = 16 * NCHUNK * 128
OUT = 128
BN_EPS = 1e-5
NEG = float(jnp.finfo(jnp.float32).min)

_HIGH = jax.lax.Precision.HIGHEST


def _dot(a, b):
    return jnp.dot(a, b, preferred_element_type=jnp.float32)


@functools.lru_cache(maxsize=None)
def _make_aggregate():
    mesh = plsc.VectorSubcoreMesh(core_axis_name="c", subcore_axis_name="s",
                                  num_cores=2, num_subcores=16)

    @functools.partial(
        pl.kernel,
        out_type=jax.ShapeDtypeStruct((2, AR, 128), jnp.float32),
        mesh=mesh,
        scratch_types=[
            pltpu.VMEM_SHARED((AR, 128), jnp.float32),
            pltpu.VMEM((64, 128), jnp.float32),
            pltpu.VMEM((SEC, 128), jnp.int32),
            pltpu.VMEM((SEC, 128), jnp.int32),
            pltpu.VMEM((128, 128), jnp.float32),
            pltpu.SemaphoreType.DMA,
        ],
    )
    def agg(y_hbm, src_hbm, dst_hbm, out_hbm, accum, zbuf, src_v, dst_v,
            rows_v, sem):
        c = lax.axis_index("c")
        s = lax.axis_index("s")
        zeros16 = jnp.zeros((16,), jnp.float32)

        def zb(i, carry):
            for k in range(8):
                zbuf[i, pl.ds(k * 16, 16)] = zeros16
            return carry

        lax.fori_loop(0, 64, zb, 0)
        for k in range(PT // 64):
            pltpu.sync_copy(zbuf, accum.at[pl.ds(s * PT + k * 64, 64)])
        plsc.subcore_barrier()

        def sec_body(g, carry):
            pltpu.sync_copy(src_hbm.at[s, pl.ds(g * SEC, SEC)], src_v)
            pltpu.sync_copy(dst_hbm.at[c, s, pl.ds(g * SEC, SEC)], dst_v)

            def body(j, carry2):
                pltpu.async_copy(y_hbm.at[src_v.at[j]], rows_v, sem).wait()
                pltpu.sync_copy(rows_v, accum.at[dst_v.at[j]], add=True)
                return carry2

            return lax.fori_loop(0, SEC, body, carry)

        lax.fori_loop(0, NCHUNK // SEC, sec_body, 0)
        plsc.subcore_barrier()
        pltpu.sync_copy(accum.at[pl.ds(s * PT, PT)],
                        out_hbm.at[c, pl.ds(s * PT, PT)])

    return agg


def _aggregate(y, src3, dst3):
    return _make_aggregate()(y, src3, dst3)


def _bn_cols(u, g, b):
    m = jnp.mean(u, axis=0)
    v = jnp.mean((u - m) * (u - m), axis=0)
    return (u - m) / jnp.sqrt(v + BN_EPS) * g + b


def _gin_layer_body(x_ref, p_ref, w1_ref, b1_ref, w2_ref, b2_ref, g_ref,
                    b_ref, out_ref):
    aggr = jnp.concatenate([p_ref[0, :H, :], p_ref[1, :N - H, :]], axis=0)
    hpre = x_ref[:N, :] + aggr
    a = jnp.maximum(_dot(hpre, w1_ref[...]) + b1_ref[...], 0.0)
    z = _dot(a, w2_ref[...]) + b2_ref[...]
    u = jnp.maximum(z, 0.0)
    un = _bn_cols(u, g_ref[...], b_ref[...])
    out_ref[...] = jnp.concatenate(
        [un, jnp.zeros((NR - N, OUT), jnp.float32)], axis=0)


def _gin_layer(x, p, w1, b1, w2, b2, bn_g, bn_b):
    return pl.pallas_call(
        _gin_layer_body,
        out_shape=jax.ShapeDtypeStruct((NR, OUT), jnp.float32),
    )(x, p, w1, b1, w2, b2, bn_g, bn_b)


def _scan_body(u_ref, batch_ref, out_ref):
    bid = batch_ref[...]
    out_ref[...] = u_ref[...]
    s = 1
    while s < NR:
        un = out_ref[...]
        u_sh = jnp.concatenate(
            [jnp.full((s, OUT), NEG, jnp.float32), un[:-s, :]], axis=0)
        b_sh = jnp.concatenate(
            [jnp.full((s, 1), -1, jnp.int32), bid[:-s, :]], axis=0)
        out_ref[...] = jnp.where(b_sh == bid, jnp.maximum(un, u_sh), un)
        s *= 2


def _pool_body(u_ref, batch_ref, fcw_ref, fcb_ref, out_ref):
    bid = batch_ref[...]
    un = u_ref[...]
    b_next = jnp.concatenate(
        [bid[1:, :], jnp.full((1, 1), -1, jnp.int32)], axis=0)
    is_end = (bid != b_next).astype(jnp.float32)
    CH = NR // 8
    gids = lax.broadcasted_iota(jnp.int32, (CH, NG), 1)
    pooled = jnp.zeros((NG, OUT), jnp.float32)
    counts = jnp.zeros((NG,), jnp.float32)
    for k in range(8):
        lo, hi = k * CH, (k + 1) * CH
        onehot = (bid[lo:hi, :] == gids).astype(jnp.float32)
        sel = onehot * is_end[lo:hi, :]
        pooled = pooled + lax.dot_general(
            sel, un[lo:hi, :], (((0,), (0,)), ((), ())),
            precision=_HIGH, preferred_element_type=jnp.float32)
        counts = counts + jnp.sum(onehot, axis=0)
    pooled = jnp.where(counts[:, None] > 0, pooled, -jnp.inf)
    out_ref[...] = jnp.maximum(_dot(pooled, fcw_ref[...]) + fcb_ref[...], 0.0)


def _pool_stage(u3, batch_pad, fc_w, fc_b):
    scanned = pl.pallas_call(
        _scan_body,
        out_shape=jax.ShapeDtypeStruct((NR, OUT), jnp.float32),
    )(u3, batch_pad)
    return pl.pallas_call(
        _pool_body,
        out_shape=jax.ShapeDtypeStruct((NG, OUT), jnp.float32),
    )(scanned, batch_pad, fc_w, fc_b)


def _branch_body(mic_ref, dis_ref, mw_ref, mb_ref, dw_ref, db_ref, mg_ref,
                 mbb_ref, dg_ref, dbb_ref, mic_out, dis_out):
    xm = jnp.maximum(_dot(mic_ref[...], mw_ref[...]) + mb_ref[...], 0.0)
    mic_out[...] = _bn_cols(xm, mg_ref[...], mbb_ref[...])
    xd = jnp.maximum(_dot(dis_ref[...], dw_ref[...]) + db_ref[...], 0.0)
    dis_out[...] = _bn_cols(xd, dg_ref[...], dbb_ref[...])


def _branches(mic_feature, dis_feature, mic_w, mic_b, dis_w, dis_b,
              bn_mic_g, bn_mic_b, bn_dis_g, bn_dis_b):
    return pl.pallas_call(
        _branch_body,
        out_shape=(jax.ShapeDtypeStruct((4096, OUT), jnp.float32),
                   jax.ShapeDtypeStruct((4096, OUT), jnp.float32)),
    )(mic_feature, dis_feature, mic_w, mic_b, dis_w, dis_b,
      bn_mic_g, bn_mic_b, bn_dis_g, bn_dis_b)


def kernel(x, edge_index, batch, mic_feature, dis_feature,
           conv1_w1, conv1_b1, conv1_w2, conv1_b2,
           conv2_w1, conv2_b1, conv2_w2, conv2_b2,
           conv3_w1, conv3_b1, conv3_w2, conv3_b2,
           bn1_g, bn1_b, bn2_g, bn2_b, bn3_g, bn3_b,
           bn_dis_g, bn_dis_b, bn_mic_g, bn_mic_b,
           fc_w, fc_b, dis_w, dis_b, mic_w, mic_b):
    pad = EP - E
    srcp = jnp.concatenate([edge_index[0], jnp.zeros((pad,), jnp.int32)])
    dstp = jnp.concatenate([edge_index[1], jnp.full((pad,), N, jnp.int32)])
    src3 = srcp.reshape(16, NCHUNK, 128)
    dummy = H + (jnp.arange(EP, dtype=jnp.int32) & 63)
    halves = []
    for c in (0, 1):
        loc = dstp - c * H
        halves.append(jnp.where((loc >= 0) & (loc < H), loc, dummy))
    dst3 = jnp.stack(halves).reshape(2, 16, NCHUNK, 128)
    x_pad = jnp.pad(x, ((0, NR - N), (0, OUT - 78)))
    w1_pad = jnp.pad(conv1_w1, ((0, OUT - 78), (0, 0)))
    batch_pad = jnp.concatenate(
        [batch, jnp.full((NR - N,), NG, jnp.int32)]).reshape(NR, 1)

    p1 = _aggregate(x_pad, src3, dst3)
    u1 = _gin_layer(x_pad, p1, w1_pad, conv1_b1, conv1_w2, conv1_b2,
                    bn1_g, bn1_b)
    p2 = _aggregate(u1, src3, dst3)
    u2 = _gin_layer(u1, p2, conv2_w1, conv2_b1, conv2_w2, conv2_b2,
                    bn2_g, bn2_b)
    p3 = _aggregate(u2, src3, dst3)
    u3 = _gin_layer(u2, p3, conv3_w1, conv3_b1, conv3_w2, conv3_b2,
                    bn3_g, bn3_b)
    x_d = _pool_stage(u3, batch_pad, fc_w, fc_b)
    x_mic, x_dis = _branches(mic_feature, dis_feature, mic_w, mic_b,
                             dis_w, dis_b, bn_mic_g, bn_mic_b,
                             bn_dis_g, bn_dis_b)
    return (x_d, x_mic, x_dis)


# speedup vs baseline: 1.9927x; 1.0543x over previous
import functools

import jax
import jax.numpy as jnp
from jax import lax
from jax.experimental import pallas as pl
from jax.experimental.pallas import tpu as pltpu
from jax.experimental.pallas import tpu_sc as plsc

N = 10000
NR = 10112
H = NR // 2
AR = 5120
PT = AR // 16
NG = 512
E = 320000
NCHUNK = 160
SEC = 8
EP = 16 * NCHUNK * 128
OUT = 128
BN_EPS = 1e-5
NEG = float(jnp.finfo(jnp.float32).min)

_HIGH = jax.lax.Precision.HIGHEST


def _dot(a, b):
    return jnp.dot(a, b, preferred_element_type=jnp.float32)


@functools.lru_cache(maxsize=None)
def _make_aggregate():
    mesh = plsc.VectorSubcoreMesh(core_axis_name="c", subcore_axis_name="s",
                                  num_cores=2, num_subcores=16)

    @functools.partial(
        pl.kernel,
        out_type=jax.ShapeDtypeStruct((2, AR, 128), jnp.float32),
        mesh=mesh,
        scratch_types=[
            pltpu.VMEM_SHARED((AR, 128), jnp.float32),
            pltpu.VMEM((SEC, 128), jnp.int32),
            pltpu.VMEM((SEC, 128), jnp.int32),
            pltpu.VMEM((128, 128), jnp.float32),
            pltpu.VMEM((128, 128), jnp.float32),
            pltpu.SemaphoreType.DMA,
            pltpu.SemaphoreType.DMA,
            pltpu.SemaphoreType.DMA,
            pltpu.SemaphoreType.DMA,
        ],
    )
    def agg(y_hbm, src_hbm, dst_hbm, out_hbm, accum, src_v, dst_v,
            rows0, rows1, semg0, semg1, sems0, sems1):
        c = lax.axis_index("c")
        s = lax.axis_index("s")
        zeros16 = jnp.zeros((16,), jnp.float32)

        def zb(i, carry):
            for k in range(8):
                rows0[i, pl.ds(k * 16, 16)] = zeros16
            return carry

        lax.fori_loop(0, 128, zb, 0)
        pltpu.sync_copy(rows0, accum.at[pl.ds(s * PT, 128)])
        pltpu.sync_copy(rows0, accum.at[pl.ds(s * PT + 128, 128)])
        pltpu.sync_copy(rows0.at[pl.ds(0, 64)],
                        accum.at[pl.ds(s * PT + 256, 64)])
        plsc.subcore_barrier()

        def drain(rows, sem):
            pltpu.make_async_copy(y_hbm.at[pl.ds(0, 128)], rows, sem).wait()

        def body(g, carry):
            @pl.when(g % (SEC // 2) == 0)
            def _():
                sec = g // (SEC // 2)
                pltpu.sync_copy(src_hbm.at[s, pl.ds(sec * SEC, SEC)], src_v)
                pltpu.sync_copy(dst_hbm.at[c, s, pl.ds(sec * SEC, SEC)], dst_v)

            q = 2 * (g % (SEC // 2))

            @pl.when(g > 0)
            def _():
                drain(rows0, sems0)
            ga = pltpu.async_copy(y_hbm.at[src_v.at[q]], rows0, semg0)

            @pl.when(g > 0)
            def _():
                drain(rows1, sems1)
            gb = pltpu.async_copy(y_hbm.at[src_v.at[q + 1]], rows1, semg1)

            ga.wait()
            pltpu.async_copy(rows0, accum.at[dst_v.at[q]], sems0, add=True)
            gb.wait()
            pltpu.async_copy(rows1, accum.at[dst_v.at[q + 1]], sems1, add=True)
            return carry

        lax.fori_loop(0, NCHUNK // 2, body, 0)
        drain(rows0, sems0)
        drain(rows1, sems1)
        plsc.subcore_barrier()
        pltpu.sync_copy(accum.at[pl.ds(s * PT, PT)],
                        out_hbm.at[c, pl.ds(s * PT, PT)])

    return agg


def _aggregate(y, src3, dst3):
    return _make_aggregate()(y, src3, dst3)


def _bn_cols(u, g, b):
    m = jnp.mean(u, axis=0)
    v = jnp.mean((u - m) * (u - m), axis=0)
    return (u - m) / jnp.sqrt(v + BN_EPS) * g + b


def _gin_layer_body(x_ref, p_ref, w1_ref, b1_ref, w2_ref, b2_ref, g_ref,
                    b_ref, out_ref):
    aggr = jnp.concatenate([p_ref[0, :H, :], p_ref[1, :N - H, :]], axis=0)
    hpre = x_ref[:N, :] + aggr
    a = jnp.maximum(_dot(hpre, w1_ref[...]) + b1_ref[...], 0.0)
    z = _dot(a, w2_ref[...]) + b2_ref[...]
    u = jnp.maximum(z, 0.0)
    un = _bn_cols(u, g_ref[...], b_ref[...])
    out_ref[...] = jnp.concatenate(
        [un, jnp.zeros((NR - N, OUT), jnp.float32)], axis=0)


def _gin_layer(x, p, w1, b1, w2, b2, bn_g, bn_b):
    return pl.pallas_call(
        _gin_layer_body,
        out_shape=jax.ShapeDtypeStruct((NR, OUT), jnp.float32),
    )(x, p, w1, b1, w2, b2, bn_g, bn_b)


def _scan_body(u_ref, batch_ref, out_ref):
    bid = batch_ref[...]
    out_ref[...] = u_ref[...]
    s = 1
    while s < NR:
        un = out_ref[...]
        u_sh = jnp.concatenate(
            [jnp.full((s, OUT), NEG, jnp.float32), un[:-s, :]], axis=0)
        b_sh = jnp.concatenate(
            [jnp.full((s, 1), -1, jnp.int32), bid[:-s, :]], axis=0)
        out_ref[...] = jnp.where(b_sh == bid, jnp.maximum(un, u_sh), un)
        s *= 2


def _pool_body(u_ref, batch_ref, fcw_ref, fcb_ref, out_ref):
    bid = batch_ref[...]
    un = u_ref[...]
    b_next = jnp.concatenate(
        [bid[1:, :], jnp.full((1, 1), -1, jnp.int32)], axis=0)
    is_end = (bid != b_next).astype(jnp.float32)
    CH = NR // 8
    gids = lax.broadcasted_iota(jnp.int32, (CH, NG), 1)
    pooled = jnp.zeros((NG, OUT), jnp.float32)
    counts = jnp.zeros((NG,), jnp.float32)
    for k in range(8):
        lo, hi = k * CH, (k + 1) * CH
        onehot = (bid[lo:hi, :] == gids).astype(jnp.float32)
        sel = onehot * is_end[lo:hi, :]
        pooled = pooled + lax.dot_general(
            sel, un[lo:hi, :], (((0,), (0,)), ((), ())),
            precision=_HIGH, preferred_element_type=jnp.float32)
        counts = counts + jnp.sum(onehot, axis=0)
    pooled = jnp.where(counts[:, None] > 0, pooled, -jnp.inf)
    out_ref[...] = jnp.maximum(_dot(pooled, fcw_ref[...]) + fcb_ref[...], 0.0)


def _pool_stage(u3, batch_pad, fc_w, fc_b):
    scanned = pl.pallas_call(
        _scan_body,
        out_shape=jax.ShapeDtypeStruct((NR, OUT), jnp.float32),
    )(u3, batch_pad)
    return pl.pallas_call(
        _pool_body,
        out_shape=jax.ShapeDtypeStruct((NG, OUT), jnp.float32),
    )(scanned, batch_pad, fc_w, fc_b)


def _branch_body(mic_ref, dis_ref, mw_ref, mb_ref, dw_ref, db_ref, mg_ref,
                 mbb_ref, dg_ref, dbb_ref, mic_out, dis_out):
    xm = jnp.maximum(_dot(mic_ref[...], mw_ref[...]) + mb_ref[...], 0.0)
    mic_out[...] = _bn_cols(xm, mg_ref[...], mbb_ref[...])
    xd = jnp.maximum(_dot(dis_ref[...], dw_ref[...]) + db_ref[...], 0.0)
    dis_out[...] = _bn_cols(xd, dg_ref[...], dbb_ref[...])


def _branches(mic_feature, dis_feature, mic_w, mic_b, dis_w, dis_b,
              bn_mic_g, bn_mic_b, bn_dis_g, bn_dis_b):
    return pl.pallas_call(
        _branch_body,
        out_shape=(jax.ShapeDtypeStruct((4096, OUT), jnp.float32),
                   jax.ShapeDtypeStruct((4096, OUT), jnp.float32)),
    )(mic_feature, dis_feature, mic_w, mic_b, dis_w, dis_b,
      bn_mic_g, bn_mic_b, bn_dis_g, bn_dis_b)


def kernel(x, edge_index, batch, mic_feature, dis_feature,
           conv1_w1, conv1_b1, conv1_w2, conv1_b2,
           conv2_w1, conv2_b1, conv2_w2, conv2_b2,
           conv3_w1, conv3_b1, conv3_w2, conv3_b2,
           bn1_g, bn1_b, bn2_g, bn2_b, bn3_g, bn3_b,
           bn_dis_g, bn_dis_b, bn_mic_g, bn_mic_b,
           fc_w, fc_b, dis_w, dis_b, mic_w, mic_b):
    pad = EP - E
    srcp = jnp.concatenate([edge_index[0], jnp.zeros((pad,), jnp.int32)])
    dstp = jnp.concatenate([edge_index[1], jnp.full((pad,), N, jnp.int32)])
    src3 = srcp.reshape(16, NCHUNK, 128)
    dummy = H + (jnp.arange(EP, dtype=jnp.int32) & 63)
    halves = []
    for c in (0, 1):
        loc = dstp - c * H
        halves.append(jnp.where((loc >= 0) & (loc < H), loc, dummy))
    dst3 = jnp.stack(halves).reshape(2, 16, NCHUNK, 128)
    x_pad = jnp.pad(x, ((0, NR - N), (0, OUT - 78)))
    w1_pad = jnp.pad(conv1_w1, ((0, OUT - 78), (0, 0)))
    batch_pad = jnp.concatenate(
        [batch, jnp.full((NR - N,), NG, jnp.int32)]).reshape(NR, 1)

    p1 = _aggregate(x_pad, src3, dst3)
    u1 = _gin_layer(x_pad, p1, w1_pad, conv1_b1, conv1_w2, conv1_b2,
                    bn1_g, bn1_b)
    p2 = _aggregate(u1, src3, dst3)
    u2 = _gin_layer(u1, p2, conv2_w1, conv2_b1, conv2_w2, conv2_b2,
                    bn2_g, bn2_b)
    p3 = _aggregate(u2, src3, dst3)
    u3 = _gin_layer(u2, p3, conv3_w1, conv3_b1, conv3_w2, conv3_b2,
                    bn3_g, bn3_b)
    x_d = _pool_stage(u3, batch_pad, fc_w, fc_b)
    x_mic, x_dis = _branches(mic_feature, dis_feature, mic_w, mic_b,
                             dis_w, dis_b, bn_mic_g, bn_mic_b,
                             bn_dis_g, bn_dis_b)
    return (x_d, x_mic, x_dis)
